# baseline (speedup 1.0000x reference)
import functools

import jax
import jax.numpy as jnp
from jax import lax
from jax.experimental import pallas as pl
from jax.experimental.pallas import tpu as pltpu
from jax.experimental.pallas import tpu_sc as plsc

N_ATOMS = 10000
N_BONDS = 160000
MAX_NB = 16
H = 256
DEPTH = 3

NC, NS, L = 2, 16, 16
NW = NC * NS

CA = 4
A_CHUNKS = 80
APW = CA * A_CHUNKS
A_PAD = APW * NW

@functools.cache
def _sc_mesh():
    return plsc.VectorSubcoreMesh(core_axis_name="c", subcore_axis_name="s")


def _gather_sum_body(msg_hbm, a2b_hbm, out_hbm, idx_v, rows_v, out_v, sem):
    wid = lax.axis_index("s") * NC + lax.axis_index("c")
    abase = wid * APW

    def chunk(c, carry):
        a0 = abase + c * CA
        pltpu.sync_copy(a2b_hbm.at[pl.ds(a0 * MAX_NB, CA * MAX_NB)], idx_v)
        pltpu.async_copy(msg_hbm.at[idx_v], rows_v, sem).wait()
        for a in range(CA):
            for j in range(H // L):
                acc = rows_v[a * MAX_NB, pl.ds(j * L, L)]
                for k in range(1, MAX_NB):
                    acc = acc + rows_v[a * MAX_NB + k, pl.ds(j * L, L)]
                out_v[a, pl.ds(j * L, L)] = acc
        pltpu.sync_copy(out_v, out_hbm.at[pl.ds(a0, CA)])
        return carry

    lax.fori_loop(0, A_CHUNKS, chunk, 0)


@functools.cache
def _gather_sum():
    return pl.kernel(
        _gather_sum_body,
        out_type=jax.ShapeDtypeStruct((A_PAD, H), jnp.float32),
        mesh=_sc_mesh(),
        scratch_types=[
            pltpu.VMEM((CA * MAX_NB,), jnp.int32),
            pltpu.VMEM((CA * MAX_NB, H), jnp.float32),
            pltpu.VMEM((CA, H), jnp.float32),
            pltpu.SemaphoreType.DMA,
        ],
    )

CB = 40
B_CHUNKS = 125
BPW = CB * B_CHUNKS


def _edge_update_body(amsg_hbm, msg_hbm, b2a_hbm, b2revb_hbm, out_hbm,
                      ia_v, ir_v, ra_v, rr_v, mv_v, sem_a, sem_r):
    wid = lax.axis_index("s") * NC + lax.axis_index("c")
    bbase = wid * BPW

    def chunk(c, carry):
        b0 = bbase + c * CB
        pltpu.sync_copy(b2a_hbm.at[pl.ds(b0, CB)], ia_v)
        pltpu.sync_copy(b2revb_hbm.at[pl.ds(b0, CB)], ir_v)
        cpa = pltpu.async_copy(amsg_hbm.at[ia_v], ra_v, sem_a)
        cpr = pltpu.async_copy(msg_hbm.at[ir_v], rr_v, sem_r)
        cpa.wait()
        cpr.wait()

        def row(r, carry2):
            for j in range(H // L):
                mv_v[r, pl.ds(j * L, L)] = (
                    ra_v[r, pl.ds(j * L, L)] - rr_v[r, pl.ds(j * L, L)])
            return carry2

        lax.fori_loop(0, CB, row, 0)
        pltpu.sync_copy(mv_v, out_hbm.at[pl.ds(b0, CB)])
        return carry

    lax.fori_loop(0, B_CHUNKS, chunk, 0)


@functools.cache
def _edge_update():
    return pl.kernel(
        _edge_update_body,
        out_type=jax.ShapeDtypeStruct((N_BONDS, H), jnp.float32),
        mesh=_sc_mesh(),
        scratch_types=[
            pltpu.VMEM((CB,), jnp.int32),
            pltpu.VMEM((CB,), jnp.int32),
            pltpu.VMEM((CB, H), jnp.float32),
            pltpu.VMEM((CB, H), jnp.float32),
            pltpu.VMEM((CB, H), jnp.float32),
            pltpu.SemaphoreType.DMA,
            pltpu.SemaphoreType.DMA,
        ],
    )

BB_A = 1600


def _proj_body(fb_ref, wi_ref, wih_ref, bih_ref, inp_ref, gi_ref):
    x = jnp.dot(fb_ref[...], wi_ref[...], preferred_element_type=jnp.float32)
    inp_ref[...] = x
    gi_ref[...] = (
        jnp.dot(x, wih_ref[...], preferred_element_type=jnp.float32)
        + bih_ref[...])


def _proj(f_bonds, w_i_t, w_ih_t, b_ih_row):
    fdim = f_bonds.shape[1]
    return pl.pallas_call(
        _proj_body,
        grid=(N_BONDS // BB_A,),
        in_specs=[
            pl.BlockSpec((BB_A, fdim), lambda i: (i, 0)),
            pl.BlockSpec((fdim, H), lambda i: (0, 0)),
            pl.BlockSpec((H, 3 * H), lambda i: (0, 0)),
            pl.BlockSpec((1, 3 * H), lambda i: (0, 0)),
        ],
        out_specs=[
            pl.BlockSpec((BB_A, H), lambda i: (i, 0)),
            pl.BlockSpec((BB_A, 3 * H), lambda i: (i, 0)),
        ],
        out_shape=[
            jax.ShapeDtypeStruct((N_BONDS, H), jnp.float32),
            jax.ShapeDtypeStruct((N_BONDS, 3 * H), jnp.float32),
        ],
    )(f_bonds, w_i_t, w_ih_t, b_ih_row)


BB_D = 1600


def _gru_body(gi_ref, m_ref, whh_ref, bhh_ref, out_ref):
    m = m_ref[...]
    gh = (jnp.dot(m, whh_ref[...], preferred_element_type=jnp.float32)
          + bhh_ref[...])
    gi = gi_ref[...]
    r = jax.nn.sigmoid(gi[:, :H] + gh[:, :H])
    z = jax.nn.sigmoid(gi[:, H:2 * H] + gh[:, H:2 * H])
    n = jnp.tanh(gi[:, 2 * H:] + r * gh[:, 2 * H:])
    out_ref[...] = (1.0 - z) * n + z * m

    @pl.when(pl.program_id(0) == 0)
    def _():
        out_ref[0:1, :] = jnp.zeros((1, H), jnp.float32)


def _gru(gi, m, w_hh_t, b_hh_row):
    return pl.pallas_call(
        _gru_body,
        grid=(N_BONDS // BB_D,),
        in_specs=[
            pl.BlockSpec((BB_D, 3 * H), lambda i: (i, 0)),
            pl.BlockSpec((BB_D, H), lambda i: (i, 0)),
            pl.BlockSpec((H, 3 * H), lambda i: (0, 0)),
            pl.BlockSpec((1, 3 * H), lambda i: (0, 0)),
        ],
        out_specs=pl.BlockSpec((BB_D, H), lambda i: (i, 0)),
        out_shape=jax.ShapeDtypeStruct((N_BONDS, H), jnp.float32),
    )(gi, m, w_hh_t, b_hh_row)


BA_E = 2000


def _out_body(fa_ref, am_ref, w1_ref, w2_ref, b_ref, mask_ref, out_ref):
    acc = jnp.dot(fa_ref[...], w1_ref[...], preferred_element_type=jnp.float32)
    acc = acc + jnp.dot(am_ref[...], w2_ref[...],
                        preferred_element_type=jnp.float32)
    acc = jnp.maximum(acc + b_ref[...], 0.0)
    out_ref[...] = acc * mask_ref[...]


def _out_proj(f_atoms, a_msg_pad, w1_t, w2_t, b_row, mask):
    return pl.pallas_call(
        _out_body,
        grid=(N_ATOMS // BA_E,),
        in_specs=[
            pl.BlockSpec((BA_E, f_atoms.shape[1]), lambda i: (i, 0)),
            pl.BlockSpec((BA_E, H), lambda i: (i, 0)),
            pl.BlockSpec((f_atoms.shape[1], H), lambda i: (0, 0)),
            pl.BlockSpec((H, H), lambda i: (0, 0)),
            pl.BlockSpec((1, H), lambda i: (0, 0)),
            pl.BlockSpec((BA_E, 1), lambda i: (i, 0)),
        ],
        out_specs=pl.BlockSpec((BA_E, H), lambda i: (i, 0)),
        out_shape=jax.ShapeDtypeStruct((N_ATOMS, H), jnp.float32),
    )(f_atoms, a_msg_pad, w1_t, w2_t, b_row, mask)


def kernel(f_atoms, f_bonds, a2b, b2a, b2revb, undirected_b2a, mask,
           W_i, W_ih, W_hh, b_ih, b_hh, W_o_w, W_o_b):
    del undirected_b2a
    afdim = f_atoms.shape[1]
    w_i_t = W_i.T
    w_ih_t = W_ih.T
    w_hh_t = W_hh.T
    w1_t = W_o_w[:, :afdim].T
    w2_t = W_o_w[:, afdim:].T

    a2b_flat = jnp.pad(a2b.reshape(-1).astype(jnp.int32),
                       (0, A_PAD * MAX_NB - N_ATOMS * MAX_NB))
    b2a = b2a.astype(jnp.int32)
    b2revb = b2revb.astype(jnp.int32)

    inp, gi = _proj(f_bonds, w_i_t, w_ih_t, b_ih.reshape(1, -1))

    msg = inp
    for _ in range(DEPTH - 1):
        amsg = _gather_sum()(msg, a2b_flat)
        m = _edge_update()(amsg, msg, b2a, b2revb)
        msg = _gru(gi, m, w_hh_t, b_hh.reshape(1, -1))

    amsg = _gather_sum()(msg, a2b_flat)
    return _out_proj(f_atoms, amsg, w1_t, w2_t, W_o_b.reshape(1, -1), mask)


# speedup vs baseline: 1.4308x; 1.4308x over previous
import functools

import jax
import jax.numpy as jnp
from jax import lax
from jax.experimental import pallas as pl
from jax.experimental.pallas import tpu as pltpu
from jax.experimental.pallas import tpu_sc as plsc

N_ATOMS = 10000
N_BONDS = 160000
MAX_NB = 16
H = 256
DEPTH = 3

NC, NS, L = 2, 16, 16
NW = NC * NS
NCOL = H // L


@functools.cache
def _sc_mesh():
    return plsc.VectorSubcoreMesh(core_axis_name="c", subcore_axis_name="s")


CA = 8
G_ROWS = CA * MAX_NB
A_CHUNKS = 40
APW = CA * A_CHUNKS
A_PAD = APW * NW


def _gather_sum_body(msg_hbm, a2b_hbm, out_hbm, idx_v,
                     rows0, rows1, out0, out1, sem0, sem1, osem0, osem1):
    wid = lax.axis_index("s") * NC + lax.axis_index("c")
    abase = wid * APW
    pltpu.sync_copy(a2b_hbm.at[pl.ds(abase * MAX_NB, APW * MAX_NB)], idx_v)

    def gsrc(c):
        return msg_hbm.at[idx_v.at[pl.ds(c * G_ROWS, G_ROWS)]]

    pltpu.async_copy(gsrc(0), rows0, sem0)

    def body(c2, carry):
        i = 2 * c2
        pltpu.async_copy(gsrc(i + 1), rows1, sem1)
        pltpu.make_async_copy(gsrc(i), rows0, sem0).wait()

        @pl.when(c2 > 0)
        def _():
            pltpu.make_async_copy(out0, out_hbm.at[pl.ds(abase, CA)],
                                  osem0).wait()

        def atom0(a, cc):
            for j in range(NCOL):
                acc = rows0[a * MAX_NB, pl.ds(j * L, L)]
                for k in range(1, MAX_NB):
                    acc = acc + rows0[a * MAX_NB + k, pl.ds(j * L, L)]
                out0[a, pl.ds(j * L, L)] = acc
            return cc

        lax.fori_loop(0, CA, atom0, 0)
        pltpu.async_copy(out0, out_hbm.at[pl.ds(abase + i * CA, CA)], osem0)

        nxt = jnp.minimum(i + 2, A_CHUNKS - 1)
        pltpu.async_copy(gsrc(nxt), rows0, sem0)
        pltpu.make_async_copy(gsrc(i + 1), rows1, sem1).wait()

        @pl.when(c2 > 0)
        def _():
            pltpu.make_async_copy(out1, out_hbm.at[pl.ds(abase, CA)],
                                  osem1).wait()

        def atom1(a, cc):
            for j in range(NCOL):
                acc = rows1[a * MAX_NB, pl.ds(j * L, L)]
                for k in range(1, MAX_NB):
                    acc = acc + rows1[a * MAX_NB + k, pl.ds(j * L, L)]
                out1[a, pl.ds(j * L, L)] = acc
            return cc

        lax.fori_loop(0, CA, atom1, 0)
        pltpu.async_copy(out1, out_hbm.at[pl.ds(abase + (i + 1) * CA, CA)],
                         osem1)
        return carry

    lax.fori_loop(0, A_CHUNKS // 2, body, 0)
    pltpu.make_async_copy(gsrc(A_CHUNKS - 1), rows0, sem0).wait()
    pltpu.make_async_copy(out0, out_hbm.at[pl.ds(abase, CA)], osem0).wait()
    pltpu.make_async_copy(out1, out_hbm.at[pl.ds(abase, CA)], osem1).wait()


@functools.cache
def _gather_sum():
    return pl.kernel(
        _gather_sum_body,
        out_type=jax.ShapeDtypeStruct((A_PAD, H), jnp.float32),
        mesh=_sc_mesh(),
        scratch_types=[
            pltpu.VMEM((APW * MAX_NB,), jnp.int32),
            pltpu.VMEM((G_ROWS, H), jnp.float32),
            pltpu.VMEM((G_ROWS, H), jnp.float32),
            pltpu.VMEM((CA, H), jnp.float32),
            pltpu.VMEM((CA, H), jnp.float32),
            pltpu.SemaphoreType.DMA,
            pltpu.SemaphoreType.DMA,
            pltpu.SemaphoreType.DMA,
            pltpu.SemaphoreType.DMA,
        ],
    )


CB = 40
B_CHUNKS = 125
BPW = CB * B_CHUNKS


def _edge_update_body(amsg_hbm, msg_hbm, b2a_hbm, b2revb_hbm, out_hbm,
                      ia_v, ir_v, ra0, ra1, rr0, rr1, mv0, mv1,
                      sa0, sa1, sr0, sr1, os0, os1):
    wid = lax.axis_index("s") * NC + lax.axis_index("c")
    bbase = wid * BPW
    pltpu.sync_copy(b2a_hbm.at[pl.ds(bbase, BPW)], ia_v)
    pltpu.sync_copy(b2revb_hbm.at[pl.ds(bbase, BPW)], ir_v)

    def asrc(c):
        return amsg_hbm.at[ia_v.at[pl.ds(c * CB, CB)]]

    def rsrc(c):
        return msg_hbm.at[ir_v.at[pl.ds(c * CB, CB)]]

    def start(c, ra, rr, sa, sr):
        pltpu.async_copy(asrc(c), ra, sa)
        pltpu.async_copy(rsrc(c), rr, sr)

    def sub(c, ra, rr, mv, osem):
        def row(r, cc):
            for j in range(NCOL):
                mv[r, pl.ds(j * L, L)] = (ra[r, pl.ds(j * L, L)]
                                          - rr[r, pl.ds(j * L, L)])
            return cc

        lax.fori_loop(0, CB, row, 0)
        pltpu.async_copy(mv, out_hbm.at[pl.ds(bbase + c * CB, CB)], osem)

    start(0, ra0, rr0, sa0, sr0)

    def body(c2, carry):
        i = 2 * c2
        start(i + 1, ra1, rr1, sa1, sr1)
        pltpu.make_async_copy(asrc(i), ra0, sa0).wait()
        pltpu.make_async_copy(rsrc(i), rr0, sr0).wait()

        @pl.when(c2 > 0)
        def _():
            pltpu.make_async_copy(mv0, out_hbm.at[pl.ds(bbase, CB)],
                                  os0).wait()

        sub(i, ra0, rr0, mv0, os0)
        start(i + 2, ra0, rr0, sa0, sr0)
        pltpu.make_async_copy(asrc(i + 1), ra1, sa1).wait()
        pltpu.make_async_copy(rsrc(i + 1), rr1, sr1).wait()

        @pl.when(c2 > 0)
        def _():
            pltpu.make_async_copy(mv1, out_hbm.at[pl.ds(bbase, CB)],
                                  os1).wait()

        sub(i + 1, ra1, rr1, mv1, os1)
        return carry

    lax.fori_loop(0, (B_CHUNKS - 1) // 2, body, 0)
    pltpu.make_async_copy(asrc(B_CHUNKS - 1), ra0, sa0).wait()
    pltpu.make_async_copy(rsrc(B_CHUNKS - 1), rr0, sr0).wait()
    pltpu.make_async_copy(mv0, out_hbm.at[pl.ds(bbase, CB)], os0).wait()
    sub(B_CHUNKS - 1, ra0, rr0, mv0, os0)
    pltpu.make_async_copy(mv0, out_hbm.at[pl.ds(bbase, CB)], os0).wait()
    pltpu.make_async_copy(mv1, out_hbm.at[pl.ds(bbase, CB)], os1).wait()


@functools.cache
def _edge_update():
    return pl.kernel(
        _edge_update_body,
        out_type=jax.ShapeDtypeStruct((N_BONDS, H), jnp.float32),
        mesh=_sc_mesh(),
        scratch_types=[
            pltpu.VMEM((BPW,), jnp.int32),
            pltpu.VMEM((BPW,), jnp.int32),
            pltpu.VMEM((CB, H), jnp.float32),
            pltpu.VMEM((CB, H), jnp.float32),
            pltpu.VMEM((CB, H), jnp.float32),
            pltpu.VMEM((CB, H), jnp.float32),
            pltpu.VMEM((CB, H), jnp.float32),
            pltpu.VMEM((CB, H), jnp.float32),
            pltpu.SemaphoreType.DMA,
            pltpu.SemaphoreType.DMA,
            pltpu.SemaphoreType.DMA,
            pltpu.SemaphoreType.DMA,
            pltpu.SemaphoreType.DMA,
            pltpu.SemaphoreType.DMA,
        ],
    )


BB_A = 1600


def _bf(x):
    return x.astype(jnp.bfloat16)


def _proj_body(fb_ref, wi_ref, wih_ref, bih_ref, inp_ref, gi_ref):
    x = jnp.dot(_bf(fb_ref[...]), wi_ref[...],
                preferred_element_type=jnp.float32)
    inp_ref[...] = x
    gi_ref[...] = (
        jnp.dot(_bf(x), wih_ref[...], preferred_element_type=jnp.float32)
        + bih_ref[...])


def _proj(f_bonds, w_i_t, w_ih_t, b_ih_row):
    fdim = f_bonds.shape[1]
    return pl.pallas_call(
        _proj_body,
        grid=(N_BONDS // BB_A,),
        in_specs=[
            pl.BlockSpec((BB_A, fdim), lambda i: (i, 0)),
            pl.BlockSpec((fdim, H), lambda i: (0, 0)),
            pl.BlockSpec((H, 3 * H), lambda i: (0, 0)),
            pl.BlockSpec((1, 3 * H), lambda i: (0, 0)),
        ],
        out_specs=[
            pl.BlockSpec((BB_A, H), lambda i: (i, 0)),
            pl.BlockSpec((BB_A, 3 * H), lambda i: (i, 0)),
        ],
        out_shape=[
            jax.ShapeDtypeStruct((N_BONDS, H), jnp.float32),
            jax.ShapeDtypeStruct((N_BONDS, 3 * H), jnp.float32),
        ],
    )(f_bonds, w_i_t, w_ih_t, b_ih_row)


BB_D = 1600


def _gru_body(gi_ref, m_ref, whh_ref, bhh_ref, out_ref):
    m = m_ref[...]
    gh = (jnp.dot(_bf(m), whh_ref[...], preferred_element_type=jnp.float32)
          + bhh_ref[...])
    gi = gi_ref[...]
    r = jax.nn.sigmoid(gi[:, :H] + gh[:, :H])
    z = jax.nn.sigmoid(gi[:, H:2 * H] + gh[:, H:2 * H])
    n = jnp.tanh(gi[:, 2 * H:] + r * gh[:, 2 * H:])
    out_ref[...] = (1.0 - z) * n + z * m

    @pl.when(pl.program_id(0) == 0)
    def _():
        out_ref[0:1, :] = jnp.zeros((1, H), jnp.float32)


def _gru(gi, m, w_hh_t, b_hh_row):
    return pl.pallas_call(
        _gru_body,
        grid=(N_BONDS // BB_D,),
        in_specs=[
            pl.BlockSpec((BB_D, 3 * H), lambda i: (i, 0)),
            pl.BlockSpec((BB_D, H), lambda i: (i, 0)),
            pl.BlockSpec((H, 3 * H), lambda i: (0, 0)),
            pl.BlockSpec((1, 3 * H), lambda i: (0, 0)),
        ],
        out_specs=pl.BlockSpec((BB_D, H), lambda i: (i, 0)),
        out_shape=jax.ShapeDtypeStruct((N_BONDS, H), jnp.float32),
    )(gi, m, w_hh_t, b_hh_row)


BA_E = 2000


def _out_body(fa_ref, am_ref, w1_ref, w2_ref, b_ref, mask_ref, out_ref):
    acc = jnp.dot(_bf(fa_ref[...]), w1_ref[...],
                  preferred_element_type=jnp.float32)
    acc = acc + jnp.dot(_bf(am_ref[...]), w2_ref[...],
                        preferred_element_type=jnp.float32)
    acc = jnp.maximum(acc + b_ref[...], 0.0)
    out_ref[...] = acc * mask_ref[...]


def _out_proj(f_atoms, a_msg_pad, w1_t, w2_t, b_row, mask):
    return pl.pallas_call(
        _out_body,
        grid=(N_ATOMS // BA_E,),
        in_specs=[
            pl.BlockSpec((BA_E, f_atoms.shape[1]), lambda i: (i, 0)),
            pl.BlockSpec((BA_E, H), lambda i: (i, 0)),
            pl.BlockSpec((f_atoms.shape[1], H), lambda i: (0, 0)),
            pl.BlockSpec((H, H), lambda i: (0, 0)),
            pl.BlockSpec((1, H), lambda i: (0, 0)),
            pl.BlockSpec((BA_E, 1), lambda i: (i, 0)),
        ],
        out_specs=pl.BlockSpec((BA_E, H), lambda i: (i, 0)),
        out_shape=jax.ShapeDtypeStruct((N_ATOMS, H), jnp.float32),
    )(f_atoms, a_msg_pad, w1_t, w2_t, b_row, mask)


def kernel(f_atoms, f_bonds, a2b, b2a, b2revb, undirected_b2a, mask,
           W_i, W_ih, W_hh, b_ih, b_hh, W_o_w, W_o_b):
    del undirected_b2a
    afdim = f_atoms.shape[1]
    w_i_t = _bf(W_i.T)
    w_ih_t = _bf(W_ih.T)
    w_hh_t = _bf(W_hh.T)
    w1_t = _bf(W_o_w[:, :afdim].T)
    w2_t = _bf(W_o_w[:, afdim:].T)

    a2b_flat = jnp.pad(a2b.reshape(-1).astype(jnp.int32),
                       (0, A_PAD * MAX_NB - N_ATOMS * MAX_NB))
    b2a = b2a.astype(jnp.int32)
    b2revb = b2revb.astype(jnp.int32)

    inp, gi = _proj(f_bonds, w_i_t, w_ih_t, b_ih.reshape(1, -1))

    msg = inp
    for _ in range(DEPTH - 1):
        amsg = _gather_sum()(msg, a2b_flat)
        m = _edge_update()(amsg, msg, b2a, b2revb)
        msg = _gru(gi, m, w_hh_t, b_hh.reshape(1, -1))

    amsg = _gather_sum()(msg, a2b_flat)
    return _out_proj(f_atoms, amsg, w1_t, w2_t, W_o_b.reshape(1, -1), mask)


# speedup vs baseline: 1.9662x; 1.3742x over previous
import functools

import jax
import jax.numpy as jnp
from jax import lax
from jax.experimental import pallas as pl
from jax.experimental.pallas import tpu as pltpu
from jax.experimental.pallas import tpu_sc as plsc

N_ATOMS = 10000
N_BONDS = 160000
MAX_NB = 16
H = 256
DEPTH = 3

NC, NS, L = 2, 16, 16
NW = NC * NS
NCOL = H // L


@functools.cache
def _sc_mesh():
    return plsc.VectorSubcoreMesh(core_axis_name="c", subcore_axis_name="s")


CA = 8
G_ROWS = CA * MAX_NB
A_CHUNKS = 40
APW = CA * A_CHUNKS
A_PAD = APW * NW


def _gather_sum_body(msg_hbm, a2b_hbm, out_hbm, idx_v,
                     rows0, rows1, out0, out1, sem0, sem1, osem0, osem1):
    wid = lax.axis_index("s") * NC + lax.axis_index("c")
    abase = wid * APW
    pltpu.sync_copy(a2b_hbm.at[pl.ds(abase * MAX_NB, APW * MAX_NB)], idx_v)

    def gsrc(c):
        return msg_hbm.at[idx_v.at[pl.ds(c * G_ROWS, G_ROWS)]]

    pltpu.async_copy(gsrc(0), rows0, sem0)

    def body(c2, carry):
        i = 2 * c2
        pltpu.async_copy(gsrc(i + 1), rows1, sem1)
        pltpu.make_async_copy(gsrc(i), rows0, sem0).wait()

        @pl.when(c2 > 0)
        def _():
            pltpu.make_async_copy(out0, out_hbm.at[pl.ds(abase, CA)],
                                  osem0).wait()

        def atom0(a, cc):
            for j in range(NCOL):
                acc = rows0[a * MAX_NB, pl.ds(j * L, L)]
                for k in range(1, MAX_NB):
                    acc = acc + rows0[a * MAX_NB + k, pl.ds(j * L, L)]
                out0[a, pl.ds(j * L, L)] = acc
            return cc

        lax.fori_loop(0, CA, atom0, 0)
        pltpu.async_copy(out0, out_hbm.at[pl.ds(abase + i * CA, CA)], osem0)

        nxt = jnp.minimum(i + 2, A_CHUNKS - 1)
        pltpu.async_copy(gsrc(nxt), rows0, sem0)
        pltpu.make_async_copy(gsrc(i + 1), rows1, sem1).wait()

        @pl.when(c2 > 0)
        def _():
            pltpu.make_async_copy(out1, out_hbm.at[pl.ds(abase, CA)],
                                  osem1).wait()

        def atom1(a, cc):
            for j in range(NCOL):
                acc = rows1[a * MAX_NB, pl.ds(j * L, L)]
                for k in range(1, MAX_NB):
                    acc = acc + rows1[a * MAX_NB + k, pl.ds(j * L, L)]
                out1[a, pl.ds(j * L, L)] = acc
            return cc

        lax.fori_loop(0, CA, atom1, 0)
        pltpu.async_copy(out1, out_hbm.at[pl.ds(abase + (i + 1) * CA, CA)],
                         osem1)
        return carry

    lax.fori_loop(0, A_CHUNKS // 2, body, 0)
    pltpu.make_async_copy(gsrc(A_CHUNKS - 1), rows0, sem0).wait()
    pltpu.make_async_copy(out0, out_hbm.at[pl.ds(abase, CA)], osem0).wait()
    pltpu.make_async_copy(out1, out_hbm.at[pl.ds(abase, CA)], osem1).wait()


@functools.cache
def _gather_sum():
    return pl.kernel(
        _gather_sum_body,
        out_type=jax.ShapeDtypeStruct((A_PAD, H), jnp.float32),
        mesh=_sc_mesh(),
        scratch_types=[
            pltpu.VMEM((APW * MAX_NB,), jnp.int32),
            pltpu.VMEM((G_ROWS, H), jnp.float32),
            pltpu.VMEM((G_ROWS, H), jnp.float32),
            pltpu.VMEM((CA, H), jnp.float32),
            pltpu.VMEM((CA, H), jnp.float32),
            pltpu.SemaphoreType.DMA,
            pltpu.SemaphoreType.DMA,
            pltpu.SemaphoreType.DMA,
            pltpu.SemaphoreType.DMA,
        ],
    )


CB = 40
B_CHUNKS = 125
BPW = CB * B_CHUNKS


def _edge_update_body(amsg_hbm, msg_hbm, b2a_hbm, b2revb_hbm, out_hbm,
                      ia_v, ir_v, ra0, ra1, rr0, rr1, mv0, mv1,
                      sa0, sa1, sr0, sr1, os0, os1):
    wid = lax.axis_index("s") * NC + lax.axis_index("c")
    bbase = wid * BPW
    pltpu.sync_copy(b2a_hbm.at[pl.ds(bbase, BPW)], ia_v)
    pltpu.sync_copy(b2revb_hbm.at[pl.ds(bbase, BPW)], ir_v)

    def asrc(c):
        return amsg_hbm.at[ia_v.at[pl.ds(c * CB, CB)]]

    def rsrc(c):
        return msg_hbm.at[ir_v.at[pl.ds(c * CB, CB)]]

    def start(c, ra, rr, sa, sr):
        pltpu.async_copy(asrc(c), ra, sa)
        pltpu.async_copy(rsrc(c), rr, sr)

    def sub(c, ra, rr, mv, osem):
        def row(r, cc):
            for j in range(NCOL):
                mv[r, pl.ds(j * L, L)] = (ra[r, pl.ds(j * L, L)]
                                          - rr[r, pl.ds(j * L, L)])
            return cc

        lax.fori_loop(0, CB, row, 0)
        pltpu.async_copy(mv, out_hbm.at[pl.ds(bbase + c * CB, CB)], osem)

    start(0, ra0, rr0, sa0, sr0)

    def body(c2, carry):
        i = 2 * c2
        start(i + 1, ra1, rr1, sa1, sr1)
        pltpu.make_async_copy(asrc(i), ra0, sa0).wait()
        pltpu.make_async_copy(rsrc(i), rr0, sr0).wait()

        @pl.when(c2 > 0)
        def _():
            pltpu.make_async_copy(mv0, out_hbm.at[pl.ds(bbase, CB)],
                                  os0).wait()

        sub(i, ra0, rr0, mv0, os0)
        start(i + 2, ra0, rr0, sa0, sr0)
        pltpu.make_async_copy(asrc(i + 1), ra1, sa1).wait()
        pltpu.make_async_copy(rsrc(i + 1), rr1, sr1).wait()

        @pl.when(c2 > 0)
        def _():
            pltpu.make_async_copy(mv1, out_hbm.at[pl.ds(bbase, CB)],
                                  os1).wait()

        sub(i + 1, ra1, rr1, mv1, os1)
        return carry

    lax.fori_loop(0, (B_CHUNKS - 1) // 2, body, 0)
    pltpu.make_async_copy(asrc(B_CHUNKS - 1), ra0, sa0).wait()
    pltpu.make_async_copy(rsrc(B_CHUNKS - 1), rr0, sr0).wait()
    pltpu.make_async_copy(mv0, out_hbm.at[pl.ds(bbase, CB)], os0).wait()
    sub(B_CHUNKS - 1, ra0, rr0, mv0, os0)
    pltpu.make_async_copy(mv0, out_hbm.at[pl.ds(bbase, CB)], os0).wait()
    pltpu.make_async_copy(mv1, out_hbm.at[pl.ds(bbase, CB)], os1).wait()


@functools.cache
def _edge_update():
    return pl.kernel(
        _edge_update_body,
        out_type=jax.ShapeDtypeStruct((N_BONDS, H), jnp.float32),
        mesh=_sc_mesh(),
        scratch_types=[
            pltpu.VMEM((BPW,), jnp.int32),
            pltpu.VMEM((BPW,), jnp.int32),
            pltpu.VMEM((CB, H), jnp.float32),
            pltpu.VMEM((CB, H), jnp.float32),
            pltpu.VMEM((CB, H), jnp.float32),
            pltpu.VMEM((CB, H), jnp.float32),
            pltpu.VMEM((CB, H), jnp.float32),
            pltpu.VMEM((CB, H), jnp.float32),
            pltpu.SemaphoreType.DMA,
            pltpu.SemaphoreType.DMA,
            pltpu.SemaphoreType.DMA,
            pltpu.SemaphoreType.DMA,
            pltpu.SemaphoreType.DMA,
            pltpu.SemaphoreType.DMA,
        ],
    )


BB_A = 1600


def _bf(x):
    return x.astype(jnp.bfloat16)


def _proj_inp_body(fb_ref, wi_ref, inp_ref):
    inp_ref[...] = jnp.dot(_bf(fb_ref[...]), wi_ref[...],
                           preferred_element_type=jnp.float32)


def _proj_inp(f_bonds, w_i_t):
    fdim = f_bonds.shape[1]
    return pl.pallas_call(
        _proj_inp_body,
        grid=(N_BONDS // BB_A,),
        in_specs=[
            pl.BlockSpec((BB_A, fdim), lambda i: (i, 0)),
            pl.BlockSpec((fdim, H), lambda i: (0, 0)),
        ],
        out_specs=pl.BlockSpec((BB_A, H), lambda i: (i, 0)),
        out_shape=jax.ShapeDtypeStruct((N_BONDS, H), jnp.float32),
    )(f_bonds, w_i_t)


def _proj_gi_body(inp_ref, wih_ref, bih_ref, gi_ref):
    gi_ref[...] = _bf(
        jnp.dot(_bf(inp_ref[...]), wih_ref[...],
                preferred_element_type=jnp.float32)
        + bih_ref[...])


def _proj_gi(inp, w_ih_t, b_ih_row):
    return pl.pallas_call(
        _proj_gi_body,
        grid=(N_BONDS // BB_A,),
        in_specs=[
            pl.BlockSpec((BB_A, H), lambda i: (i, 0)),
            pl.BlockSpec((H, 3 * H), lambda i: (0, 0)),
            pl.BlockSpec((1, 3 * H), lambda i: (0, 0)),
        ],
        out_specs=pl.BlockSpec((BB_A, 3 * H), lambda i: (i, 0)),
        out_shape=jax.ShapeDtypeStruct((N_BONDS, 3 * H), jnp.bfloat16),
    )(inp, w_ih_t, b_ih_row)


BB_D = 1600


def _gru_body(gi_ref, m_ref, whh_ref, bhh_ref, out_ref):
    m = m_ref[...]
    gh = (jnp.dot(_bf(m), whh_ref[...], preferred_element_type=jnp.float32)
          + bhh_ref[...])
    gi = gi_ref[...].astype(jnp.float32)
    r = jax.nn.sigmoid(gi[:, :H] + gh[:, :H])
    z = jax.nn.sigmoid(gi[:, H:2 * H] + gh[:, H:2 * H])
    n = jnp.tanh(gi[:, 2 * H:] + r * gh[:, 2 * H:])
    out_ref[...] = (1.0 - z) * n + z * m

    @pl.when(pl.program_id(0) == 0)
    def _():
        out_ref[0:1, :] = jnp.zeros((1, H), jnp.float32)


def _gru(gi, m, w_hh_t, b_hh_row):
    return pl.pallas_call(
        _gru_body,
        grid=(N_BONDS // BB_D,),
        in_specs=[
            pl.BlockSpec((BB_D, 3 * H), lambda i: (i, 0)),
            pl.BlockSpec((BB_D, H), lambda i: (i, 0)),
            pl.BlockSpec((H, 3 * H), lambda i: (0, 0)),
            pl.BlockSpec((1, 3 * H), lambda i: (0, 0)),
        ],
        out_specs=pl.BlockSpec((BB_D, H), lambda i: (i, 0)),
        out_shape=jax.ShapeDtypeStruct((N_BONDS, H), jnp.float32),
    )(gi, m, w_hh_t, b_hh_row)


BA_E = 2000


def _out_body(fa_ref, am_ref, w1_ref, w2_ref, b_ref, mask_ref, out_ref):
    acc = jnp.dot(_bf(fa_ref[...]), w1_ref[...],
                  preferred_element_type=jnp.float32)
    acc = acc + jnp.dot(_bf(am_ref[...]), w2_ref[...],
                        preferred_element_type=jnp.float32)
    acc = jnp.maximum(acc + b_ref[...], 0.0)
    out_ref[...] = acc * mask_ref[...]


def _out_proj(f_atoms, a_msg_pad, w1_t, w2_t, b_row, mask):
    return pl.pallas_call(
        _out_body,
        grid=(N_ATOMS // BA_E,),
        in_specs=[
            pl.BlockSpec((BA_E, f_atoms.shape[1]), lambda i: (i, 0)),
            pl.BlockSpec((BA_E, H), lambda i: (i, 0)),
            pl.BlockSpec((f_atoms.shape[1], H), lambda i: (0, 0)),
            pl.BlockSpec((H, H), lambda i: (0, 0)),
            pl.BlockSpec((1, H), lambda i: (0, 0)),
            pl.BlockSpec((BA_E, 1), lambda i: (i, 0)),
        ],
        out_specs=pl.BlockSpec((BA_E, H), lambda i: (i, 0)),
        out_shape=jax.ShapeDtypeStruct((N_ATOMS, H), jnp.float32),
    )(f_atoms, a_msg_pad, w1_t, w2_t, b_row, mask)


def kernel(f_atoms, f_bonds, a2b, b2a, b2revb, undirected_b2a, mask,
           W_i, W_ih, W_hh, b_ih, b_hh, W_o_w, W_o_b):
    del undirected_b2a
    afdim = f_atoms.shape[1]
    w_i_t = _bf(W_i.T)
    w_ih_t = _bf(W_ih.T)
    w_hh_t = _bf(W_hh.T)
    w1_t = _bf(W_o_w[:, :afdim].T)
    w2_t = _bf(W_o_w[:, afdim:].T)

    n_pad = A_PAD * MAX_NB - N_ATOMS * MAX_NB
    a2b_flat = jnp.concatenate([
        a2b.reshape(-1).astype(jnp.int32),
        jnp.arange(n_pad, dtype=jnp.int32),
    ])
    b2a = b2a.astype(jnp.int32)
    b2revb = b2revb.astype(jnp.int32)

    inp = _proj_inp(f_bonds, w_i_t)
    gi = _proj_gi(inp, w_ih_t, b_ih.reshape(1, -1))

    msg = inp
    for _ in range(DEPTH - 1):
        amsg = _gather_sum()(msg, a2b_flat)
        m = _edge_update()(amsg, msg, b2a, b2revb)
        msg = _gru(gi, m, w_hh_t, b_hh.reshape(1, -1))

    amsg = _gather_sum()(msg, a2b_flat)
    return _out_proj(f_atoms, amsg, w1_t, w2_t, W_o_b.reshape(1, -1), mask)


# speedup vs baseline: 2.1085x; 1.0724x over previous
import functools

import jax
import jax.numpy as jnp
from jax import lax
from jax.experimental import pallas as pl
from jax.experimental.pallas import tpu as pltpu
from jax.experimental.pallas import tpu_sc as plsc

N_ATOMS = 10000
N_BONDS = 160000
MAX_NB = 16
H = 256
DEPTH = 3

NC, NS, L = 2, 16, 16
NW = NC * NS
NCOL = H // L


@functools.cache
def _sc_mesh():
    return plsc.VectorSubcoreMesh(core_axis_name="c", subcore_axis_name="s")


CA = 8
G_ROWS = CA * MAX_NB
A_CHUNKS = 40
APW = CA * A_CHUNKS
A_PAD = APW * NW


def _gather_sum_body(msg_hbm, a2b_hbm, out_hbm, idx_v,
                     rows0, rows1, out0, out1, sem0, sem1, osem0, osem1):
    wid = lax.axis_index("s") * NC + lax.axis_index("c")
    abase = wid * APW
    pltpu.sync_copy(a2b_hbm.at[pl.ds(abase * MAX_NB, APW * MAX_NB)], idx_v)

    def gsrc(c):
        return msg_hbm.at[idx_v.at[pl.ds(c * G_ROWS, G_ROWS)]]

    pltpu.async_copy(gsrc(0), rows0, sem0)

    def body(c2, carry):
        i = 2 * c2
        pltpu.async_copy(gsrc(i + 1), rows1, sem1)
        pltpu.make_async_copy(gsrc(i), rows0, sem0).wait()

        @pl.when(c2 > 0)
        def _():
            pltpu.make_async_copy(out0, out_hbm.at[pl.ds(abase, CA)],
                                  osem0).wait()

        def atom0(a, cc):
            for j in range(NCOL):
                acc = rows0[a * MAX_NB, pl.ds(j * L, L)]
                for k in range(1, MAX_NB):
                    acc = acc + rows0[a * MAX_NB + k, pl.ds(j * L, L)]
                out0[a, pl.ds(j * L, L)] = acc
            return cc

        lax.fori_loop(0, CA, atom0, 0)
        pltpu.async_copy(out0, out_hbm.at[pl.ds(abase + i * CA, CA)], osem0)

        nxt = jnp.minimum(i + 2, A_CHUNKS - 1)
        pltpu.async_copy(gsrc(nxt), rows0, sem0)
        pltpu.make_async_copy(gsrc(i + 1), rows1, sem1).wait()

        @pl.when(c2 > 0)
        def _():
            pltpu.make_async_copy(out1, out_hbm.at[pl.ds(abase, CA)],
                                  osem1).wait()

        def atom1(a, cc):
            for j in range(NCOL):
                acc = rows1[a * MAX_NB, pl.ds(j * L, L)]
                for k in range(1, MAX_NB):
                    acc = acc + rows1[a * MAX_NB + k, pl.ds(j * L, L)]
                out1[a, pl.ds(j * L, L)] = acc
            return cc

        lax.fori_loop(0, CA, atom1, 0)
        pltpu.async_copy(out1, out_hbm.at[pl.ds(abase + (i + 1) * CA, CA)],
                         osem1)
        return carry

    lax.fori_loop(0, A_CHUNKS // 2, body, 0)
    pltpu.make_async_copy(gsrc(A_CHUNKS - 1), rows0, sem0).wait()
    pltpu.make_async_copy(out0, out_hbm.at[pl.ds(abase, CA)], osem0).wait()
    pltpu.make_async_copy(out1, out_hbm.at[pl.ds(abase, CA)], osem1).wait()


@functools.cache
def _gather_sum():
    return pl.kernel(
        _gather_sum_body,
        out_type=jax.ShapeDtypeStruct((A_PAD, H), jnp.float32),
        mesh=_sc_mesh(),
        scratch_types=[
            pltpu.VMEM((APW * MAX_NB,), jnp.int32),
            pltpu.VMEM((G_ROWS, H), jnp.float32),
            pltpu.VMEM((G_ROWS, H), jnp.float32),
            pltpu.VMEM((CA, H), jnp.float32),
            pltpu.VMEM((CA, H), jnp.float32),
            pltpu.SemaphoreType.DMA,
            pltpu.SemaphoreType.DMA,
            pltpu.SemaphoreType.DMA,
            pltpu.SemaphoreType.DMA,
        ],
    )


CB = 40
B_CHUNKS = 125
BPW = CB * B_CHUNKS


def _edge_update_body(amsg_hbm, msg_hbm, b2a_hbm, b2revb_hbm, out_hbm,
                      ia_v, ir_v, ra0, ra1, rr0, rr1, mv0, mv1,
                      sa0, sa1, sr0, sr1, os0, os1):
    wid = lax.axis_index("s") * NC + lax.axis_index("c")
    bbase = wid * BPW
    pltpu.sync_copy(b2a_hbm.at[pl.ds(bbase, BPW)], ia_v)
    pltpu.sync_copy(b2revb_hbm.at[pl.ds(bbase, BPW)], ir_v)

    def asrc(c):
        return amsg_hbm.at[ia_v.at[pl.ds(c * CB, CB)]]

    def rsrc(c):
        return msg_hbm.at[ir_v.at[pl.ds(c * CB, CB)]]

    def start(c, ra, rr, sa, sr):
        pltpu.async_copy(asrc(c), ra, sa)
        pltpu.async_copy(rsrc(c), rr, sr)

    def sub(c, ra, rr, mv, osem):
        def row(r, cc):
            for j in range(NCOL):
                mv[r, pl.ds(j * L, L)] = (ra[r, pl.ds(j * L, L)]
                                          - rr[r, pl.ds(j * L, L)])
            return cc

        lax.fori_loop(0, CB, row, 0)
        pltpu.async_copy(mv, out_hbm.at[pl.ds(bbase + c * CB, CB)], osem)

    start(0, ra0, rr0, sa0, sr0)

    def body(c2, carry):
        i = 2 * c2
        start(i + 1, ra1, rr1, sa1, sr1)
        pltpu.make_async_copy(asrc(i), ra0, sa0).wait()
        pltpu.make_async_copy(rsrc(i), rr0, sr0).wait()

        @pl.when(c2 > 0)
        def _():
            pltpu.make_async_copy(mv0, out_hbm.at[pl.ds(bbase, CB)],
                                  os0).wait()

        sub(i, ra0, rr0, mv0, os0)
        start(i + 2, ra0, rr0, sa0, sr0)
        pltpu.make_async_copy(asrc(i + 1), ra1, sa1).wait()
        pltpu.make_async_copy(rsrc(i + 1), rr1, sr1).wait()

        @pl.when(c2 > 0)
        def _():
            pltpu.make_async_copy(mv1, out_hbm.at[pl.ds(bbase, CB)],
                                  os1).wait()

        sub(i + 1, ra1, rr1, mv1, os1)
        return carry

    lax.fori_loop(0, (B_CHUNKS - 1) // 2, body, 0)
    pltpu.make_async_copy(asrc(B_CHUNKS - 1), ra0, sa0).wait()
    pltpu.make_async_copy(rsrc(B_CHUNKS - 1), rr0, sr0).wait()
    pltpu.make_async_copy(mv0, out_hbm.at[pl.ds(bbase, CB)], os0).wait()
    sub(B_CHUNKS - 1, ra0, rr0, mv0, os0)
    pltpu.make_async_copy(mv0, out_hbm.at[pl.ds(bbase, CB)], os0).wait()
    pltpu.make_async_copy(mv1, out_hbm.at[pl.ds(bbase, CB)], os1).wait()


@functools.cache
def _edge_update():
    return pl.kernel(
        _edge_update_body,
        out_type=jax.ShapeDtypeStruct((N_BONDS, H), jnp.float32),
        mesh=_sc_mesh(),
        scratch_types=[
            pltpu.VMEM((BPW,), jnp.int32),
            pltpu.VMEM((BPW,), jnp.int32),
            pltpu.VMEM((CB, H), jnp.float32),
            pltpu.VMEM((CB, H), jnp.float32),
            pltpu.VMEM((CB, H), jnp.float32),
            pltpu.VMEM((CB, H), jnp.float32),
            pltpu.VMEM((CB, H), jnp.float32),
            pltpu.VMEM((CB, H), jnp.float32),
            pltpu.SemaphoreType.DMA,
            pltpu.SemaphoreType.DMA,
            pltpu.SemaphoreType.DMA,
            pltpu.SemaphoreType.DMA,
            pltpu.SemaphoreType.DMA,
            pltpu.SemaphoreType.DMA,
        ],
    )


BB_A = 1600


def _bf(x):
    return x.astype(jnp.bfloat16)


def _proj_inp_body(fb_ref, wi_ref, inp_ref):
    inp_ref[...] = jnp.dot(_bf(fb_ref[...]), wi_ref[...],
                           preferred_element_type=jnp.float32)


def _proj_inp(f_bonds, w_i_t):
    fdim = f_bonds.shape[1]
    return pl.pallas_call(
        _proj_inp_body,
        grid=(N_BONDS // BB_A,),
        in_specs=[
            pl.BlockSpec((BB_A, fdim), lambda i: (i, 0)),
            pl.BlockSpec((fdim, H), lambda i: (0, 0)),
        ],
        out_specs=pl.BlockSpec((BB_A, H), lambda i: (i, 0)),
        out_shape=jax.ShapeDtypeStruct((N_BONDS, H), jnp.float32),
    )(f_bonds, w_i_t)


BB_D = 1600


def _gru_body(inp_ref, m_ref, wih_ref, whh_ref, bih_ref, bhh_ref, out_ref):
    m = m_ref[...]
    gi = (jnp.dot(_bf(inp_ref[...]), wih_ref[...],
                  preferred_element_type=jnp.float32) + bih_ref[...])
    gh = (jnp.dot(_bf(m), whh_ref[...], preferred_element_type=jnp.float32)
          + bhh_ref[...])
    r = jax.nn.sigmoid(gi[:, :H] + gh[:, :H])
    z = jax.nn.sigmoid(gi[:, H:2 * H] + gh[:, H:2 * H])
    n = jnp.tanh(gi[:, 2 * H:] + r * gh[:, 2 * H:])
    out_ref[...] = (1.0 - z) * n + z * m

    @pl.when(pl.program_id(0) == 0)
    def _():
        out_ref[0:1, :] = jnp.zeros((1, H), jnp.float32)


def _gru(inp, m, w_ih_t, w_hh_t, b_ih_row, b_hh_row):
    return pl.pallas_call(
        _gru_body,
        grid=(N_BONDS // BB_D,),
        in_specs=[
            pl.BlockSpec((BB_D, H), lambda i: (i, 0)),
            pl.BlockSpec((BB_D, H), lambda i: (i, 0)),
            pl.BlockSpec((H, 3 * H), lambda i: (0, 0)),
            pl.BlockSpec((H, 3 * H), lambda i: (0, 0)),
            pl.BlockSpec((1, 3 * H), lambda i: (0, 0)),
            pl.BlockSpec((1, 3 * H), lambda i: (0, 0)),
        ],
        out_specs=pl.BlockSpec((BB_D, H), lambda i: (i, 0)),
        out_shape=jax.ShapeDtypeStruct((N_BONDS, H), jnp.float32),
    )(inp, m, w_ih_t, w_hh_t, b_ih_row, b_hh_row)


BA_E = 2000


def _out_body(fa_ref, am_ref, w1_ref, w2_ref, b_ref, mask_ref, out_ref):
    acc = jnp.dot(_bf(fa_ref[...]), w1_ref[...],
                  preferred_element_type=jnp.float32)
    acc = acc + jnp.dot(_bf(am_ref[...]), w2_ref[...],
                        preferred_element_type=jnp.float32)
    acc = jnp.maximum(acc + b_ref[...], 0.0)
    out_ref[...] = acc * mask_ref[...]


def _out_proj(f_atoms, a_msg_pad, w1_t, w2_t, b_row, mask):
    return pl.pallas_call(
        _out_body,
        grid=(N_ATOMS // BA_E,),
        in_specs=[
            pl.BlockSpec((BA_E, f_atoms.shape[1]), lambda i: (i, 0)),
            pl.BlockSpec((BA_E, H), lambda i: (i, 0)),
            pl.BlockSpec((f_atoms.shape[1], H), lambda i: (0, 0)),
            pl.BlockSpec((H, H), lambda i: (0, 0)),
            pl.BlockSpec((1, H), lambda i: (0, 0)),
            pl.BlockSpec((BA_E, 1), lambda i: (i, 0)),
        ],
        out_specs=pl.BlockSpec((BA_E, H), lambda i: (i, 0)),
        out_shape=jax.ShapeDtypeStruct((N_ATOMS, H), jnp.float32),
    )(f_atoms, a_msg_pad, w1_t, w2_t, b_row, mask)


def kernel(f_atoms, f_bonds, a2b, b2a, b2revb, undirected_b2a, mask,
           W_i, W_ih, W_hh, b_ih, b_hh, W_o_w, W_o_b):
    del undirected_b2a
    afdim = f_atoms.shape[1]
    w_i_t = _bf(W_i.T)
    w_ih_t = _bf(W_ih.T)
    w_hh_t = _bf(W_hh.T)
    w1_t = _bf(W_o_w[:, :afdim].T)
    w2_t = _bf(W_o_w[:, afdim:].T)

    n_pad = A_PAD * MAX_NB - N_ATOMS * MAX_NB
    a2b_flat = jnp.concatenate([
        a2b.reshape(-1).astype(jnp.int32),
        jnp.arange(n_pad, dtype=jnp.int32),
    ])
    b2a = b2a.astype(jnp.int32)
    b2revb = b2revb.astype(jnp.int32)

    inp = _proj_inp(f_bonds, w_i_t)

    msg = inp
    for _ in range(DEPTH - 1):
        amsg = _gather_sum()(msg, a2b_flat)
        m = _edge_update()(amsg, msg, b2a, b2revb)
        msg = _gru(inp, m, w_ih_t, w_hh_t,
                   b_ih.reshape(1, -1), b_hh.reshape(1, -1))

    amsg = _gather_sum()(msg, a2b_flat)
    return _out_proj(f_atoms, amsg, w1_t, w2_t, W_o_b.reshape(1, -1), mask)
